# Initial kernel scaffold; baseline (speedup 1.0000x reference)
import jax
import jax.numpy as jnp
from jax.experimental import pallas as pl


def kernel(x, edge_index, edge_attr, W, b):
    raise NotImplementedError("write your pallas kernel here")


# speedup vs baseline: 4.1766x; 4.1766x over previous
import jax
import jax.numpy as jnp
from jax import lax
from jax.experimental import pallas as pl
from jax.experimental.pallas import tpu as pltpu
from jax.experimental.pallas import tpu_sc as plsc

N_NODES = 10000
E_EDGES = 320000
D_FEAT = 128
D_EDGE = 16
D_OUT = 128

NUM_CORES = 2
NUM_SUBCORES = 16
NUM_TILES = NUM_CORES * NUM_SUBCORES
GROUP = 128
EDGES_PER_TILE = 10240
E_PAD = EDGES_PER_TILE * NUM_TILES
ROWS_PER_TILE = EDGES_PER_TILE // GROUP
CHUNK_ROWS = 16
CHUNK_EDGES = CHUNK_ROWS * GROUP
NUM_CHUNKS = ROWS_PER_TILE // CHUNK_ROWS
TBL_ROWS = 10240
STRIPE = TBL_ROWS // NUM_SUBCORES
DUMMY_ROW = N_NODES


def _sc_scatter(idx_hbm, attr_hbm, ones_hbm, zeros_hbm,
                sums_out, counts_out,
                idx_v, data_v, ones_v, sums_sh, counts_sh):
    c = lax.axis_index("c")
    s = lax.axis_index("s")
    t = c * NUM_SUBCORES + s

    pltpu.sync_copy(ones_hbm, ones_v)
    pltpu.sync_copy(zeros_hbm, sums_sh.at[pl.ds(s * STRIPE, STRIPE)])
    pltpu.sync_copy(zeros_hbm, counts_sh.at[pl.ds(s * STRIPE, STRIPE)])
    plsc.subcore_barrier()

    def chunk_body(ch, carry):
        row0 = t * ROWS_PER_TILE + ch * CHUNK_ROWS
        pltpu.sync_copy(idx_hbm.at[pl.ds(row0, CHUNK_ROWS)], idx_v)
        pltpu.sync_copy(attr_hbm.at[pl.ds(row0, CHUNK_ROWS)], data_v)

        def group_body(j, carry2):
            idx_row = idx_v.at[j]
            pltpu.sync_copy(data_v.at[j], sums_sh.at[idx_row], add=True)
            pltpu.sync_copy(ones_v, counts_sh.at[idx_row], add=True)
            return carry2

        lax.fori_loop(0, CHUNK_ROWS, group_body, 0)
        return carry

    lax.fori_loop(0, NUM_CHUNKS, chunk_body, 0)

    plsc.subcore_barrier()
    pltpu.sync_copy(sums_sh.at[pl.ds(s * STRIPE, STRIPE)],
                    sums_out.at[c].at[pl.ds(s * STRIPE, STRIPE)])
    pltpu.sync_copy(counts_sh.at[pl.ds(s * STRIPE, STRIPE)],
                    counts_out.at[c].at[pl.ds(s * STRIPE, STRIPE)])


def _tc_combine(x_ref, sums_ref, counts_ref, wx_ref, we_ref, b_ref, out_ref):
    ssum = sums_ref[0] + sums_ref[1]
    cnt = counts_ref[0, :, 0:1] + counts_ref[1, :, 0:1]
    recv = ssum / jnp.maximum(cnt, 1.0)
    acc = jnp.dot(x_ref[...], wx_ref[...], preferred_element_type=jnp.float32)
    acc = acc + jnp.dot(recv, we_ref[...], preferred_element_type=jnp.float32)
    out_ref[...] = acc + b_ref[...]


BLK = 2000


def kernel(x, edge_index, edge_attr, W, b):
    dst = edge_index[1].astype(jnp.int32)
    pad = E_PAD - E_EDGES
    dst_p = jnp.concatenate(
        [dst, jnp.full((pad,), DUMMY_ROW, jnp.int32)]
    ).reshape(NUM_TILES * ROWS_PER_TILE, GROUP)
    attr_p = jnp.concatenate(
        [edge_attr.astype(jnp.float32),
         jnp.zeros((pad, D_EDGE), jnp.float32)]
    ).reshape(NUM_TILES * ROWS_PER_TILE, GROUP, D_EDGE)
    ones = jnp.ones((GROUP, D_EDGE), jnp.float32)
    zeros = jnp.zeros((STRIPE, D_EDGE), jnp.float32)

    mesh = plsc.VectorSubcoreMesh(core_axis_name="c", subcore_axis_name="s")
    sums, counts = pl.kernel(
        _sc_scatter,
        mesh=mesh,
        out_type=[
            jax.ShapeDtypeStruct((NUM_CORES, TBL_ROWS, D_EDGE), jnp.float32),
            jax.ShapeDtypeStruct((NUM_CORES, TBL_ROWS, D_EDGE), jnp.float32),
        ],
        scratch_types=[
            pltpu.VMEM((CHUNK_ROWS, GROUP), jnp.int32),
            pltpu.VMEM((CHUNK_ROWS, GROUP, D_EDGE), jnp.float32),
            pltpu.VMEM((GROUP, D_EDGE), jnp.float32),
            pltpu.VMEM_SHARED((TBL_ROWS, D_EDGE), jnp.float32),
            pltpu.VMEM_SHARED((TBL_ROWS, D_EDGE), jnp.float32),
        ],
        compiler_params=pltpu.CompilerParams(use_tc_tiling_on_sc=False),
    )(dst_p, attr_p, ones, zeros)

    wx = W[:D_FEAT]
    we = W[D_FEAT:]
    b2 = b.reshape(1, D_OUT)
    out = pl.pallas_call(
        _tc_combine,
        grid=(N_NODES // BLK,),
        in_specs=[
            pl.BlockSpec((BLK, D_FEAT), lambda i: (i, 0)),
            pl.BlockSpec((NUM_CORES, BLK, D_EDGE), lambda i: (0, i, 0)),
            pl.BlockSpec((NUM_CORES, BLK, D_EDGE), lambda i: (0, i, 0)),
            pl.BlockSpec((D_FEAT, D_OUT), lambda i: (0, 0)),
            pl.BlockSpec((D_EDGE, D_OUT), lambda i: (0, 0)),
            pl.BlockSpec((1, D_OUT), lambda i: (0, 0)),
        ],
        out_specs=pl.BlockSpec((BLK, D_OUT), lambda i: (i, 0)),
        out_shape=jax.ShapeDtypeStruct((N_NODES, D_OUT), jnp.float32),
    )(x, sums, counts, wx, we, b2)
    return out


# speedup vs baseline: 6.3053x; 1.5097x over previous
import jax
import jax.numpy as jnp
from jax import lax
from jax.experimental import pallas as pl
from jax.experimental.pallas import tpu as pltpu
from jax.experimental.pallas import tpu_sc as plsc

N_NODES = 10000
E_EDGES = 320000
D_FEAT = 128
D_EDGE = 16
D_OUT = 128

NUM_CORES = 2
NUM_SUBCORES = 16
NUM_TILES = NUM_CORES * NUM_SUBCORES
GROUP = 128
NUM_ROWS = E_EDGES // GROUP
ROWS_TILE = NUM_ROWS // NUM_TILES
EXTRA_BASE = ROWS_TILE * NUM_TILES
NUM_EXTRA = NUM_ROWS - EXTRA_BASE
CHUNK_ROWS = 16
FULL_CHUNKS = ROWS_TILE // CHUNK_ROWS
TAIL_ROWS = ROWS_TILE - FULL_CHUNKS * CHUNK_ROWS
TBL_ROWS = 10240
STRIPE = TBL_ROWS // NUM_SUBCORES


def _sc_scatter(idx_hbm, attr_hbm, ones_hbm, zeros_hbm, zeros1_hbm,
                sums_out, counts_out,
                idx_v, data_v, ones_v, sums_sh, counts_sh):
    c = lax.axis_index("c")
    s = lax.axis_index("s")
    t = c * NUM_SUBCORES + s

    pltpu.sync_copy(ones_hbm, ones_v)
    pltpu.sync_copy(zeros_hbm, sums_sh.at[pl.ds(s * STRIPE, STRIPE)])
    pltpu.sync_copy(zeros1_hbm, counts_sh.at[pl.ds(s * STRIPE, STRIPE)])
    plsc.subcore_barrier()

    row0 = t * ROWS_TILE

    def scatter_groups(n_rows):
        def group_body(j, carry2):
            idx_row = idx_v.at[j]
            pltpu.sync_copy(data_v.at[j], sums_sh.at[idx_row], add=True)
            pltpu.sync_copy(ones_v, counts_sh.at[idx_row], add=True)
            return carry2
        lax.fori_loop(0, n_rows, group_body, 0)

    def chunk_body(ch, carry):
        r = row0 + ch * CHUNK_ROWS
        pltpu.sync_copy(idx_hbm.at[pl.ds(r, CHUNK_ROWS)], idx_v)
        pltpu.sync_copy(attr_hbm.at[pl.ds(r, CHUNK_ROWS)], data_v)
        scatter_groups(CHUNK_ROWS)
        return carry

    lax.fori_loop(0, FULL_CHUNKS, chunk_body, 0)

    r_tail = row0 + FULL_CHUNKS * CHUNK_ROWS
    pltpu.sync_copy(idx_hbm.at[pl.ds(r_tail, TAIL_ROWS)],
                    idx_v.at[pl.ds(0, TAIL_ROWS)])
    pltpu.sync_copy(attr_hbm.at[pl.ds(r_tail, TAIL_ROWS)],
                    data_v.at[pl.ds(0, TAIL_ROWS)])
    scatter_groups(TAIL_ROWS)

    @pl.when(t < NUM_EXTRA)
    def _():
        r_x = EXTRA_BASE + t
        pltpu.sync_copy(idx_hbm.at[pl.ds(r_x, 1)], idx_v.at[pl.ds(0, 1)])
        pltpu.sync_copy(attr_hbm.at[pl.ds(r_x, 1)], data_v.at[pl.ds(0, 1)])
        scatter_groups(1)

    plsc.subcore_barrier()
    pltpu.sync_copy(sums_sh.at[pl.ds(s * STRIPE, STRIPE)],
                    sums_out.at[c].at[pl.ds(s * STRIPE, STRIPE)])
    pltpu.sync_copy(counts_sh.at[pl.ds(s * STRIPE, STRIPE)],
                    counts_out.at[c].at[pl.ds(s * STRIPE, STRIPE)])


def _tc_combine(x_ref, sums_ref, counts_ref, wx_ref, we_ref, b_ref, out_ref):
    ssum = sums_ref[0] + sums_ref[1]
    cnt = counts_ref[0, :, 0:1] + counts_ref[1, :, 0:1]
    recv = ssum / jnp.maximum(cnt, 1.0)
    acc = jnp.dot(x_ref[...], wx_ref[...], preferred_element_type=jnp.float32)
    acc = acc + jnp.dot(recv, we_ref[...], preferred_element_type=jnp.float32)
    out_ref[...] = acc + b_ref[...]


BLK = 2000


def kernel(x, edge_index, edge_attr, W, b):
    dst = edge_index[1].astype(jnp.int32)
    dst_p = dst.reshape(NUM_ROWS, GROUP)
    attr_p = edge_attr.astype(jnp.float32).reshape(NUM_ROWS, GROUP, D_EDGE)
    ones = jnp.ones((GROUP, D_EDGE), jnp.float32)
    zeros = jnp.zeros((STRIPE, D_EDGE), jnp.float32)
    zeros1 = jnp.zeros((STRIPE, D_EDGE), jnp.float32)

    mesh = plsc.VectorSubcoreMesh(core_axis_name="c", subcore_axis_name="s")
    sums, counts = pl.kernel(
        _sc_scatter,
        mesh=mesh,
        out_type=[
            jax.ShapeDtypeStruct((NUM_CORES, TBL_ROWS, D_EDGE), jnp.float32),
            jax.ShapeDtypeStruct((NUM_CORES, TBL_ROWS, D_EDGE), jnp.float32),
        ],
        scratch_types=[
            pltpu.VMEM((CHUNK_ROWS, GROUP), jnp.int32),
            pltpu.VMEM((CHUNK_ROWS, GROUP, D_EDGE), jnp.float32),
            pltpu.VMEM((GROUP, D_EDGE), jnp.float32),
            pltpu.VMEM_SHARED((TBL_ROWS, D_EDGE), jnp.float32),
            pltpu.VMEM_SHARED((TBL_ROWS, D_EDGE), jnp.float32),
        ],
        compiler_params=pltpu.CompilerParams(use_tc_tiling_on_sc=False),
    )(dst_p, attr_p, ones, zeros, zeros1)

    wx = W[:D_FEAT]
    we = W[D_FEAT:]
    b2 = b.reshape(1, D_OUT)
    out = pl.pallas_call(
        _tc_combine,
        grid=(N_NODES // BLK,),
        in_specs=[
            pl.BlockSpec((BLK, D_FEAT), lambda i: (i, 0)),
            pl.BlockSpec((NUM_CORES, BLK, D_EDGE), lambda i: (0, i, 0)),
            pl.BlockSpec((NUM_CORES, BLK, D_EDGE), lambda i: (0, i, 0)),
            pl.BlockSpec((D_FEAT, D_OUT), lambda i: (0, 0)),
            pl.BlockSpec((D_EDGE, D_OUT), lambda i: (0, 0)),
            pl.BlockSpec((1, D_OUT), lambda i: (0, 0)),
        ],
        out_specs=pl.BlockSpec((BLK, D_OUT), lambda i: (i, 0)),
        out_shape=jax.ShapeDtypeStruct((N_NODES, D_OUT), jnp.float32),
    )(x, sums, counts, wx, we, b2)
    return out
